# baseline (speedup 1.0000x reference)
import functools

import jax
import jax.numpy as jnp
from jax import lax
from jax.experimental import pallas as pl
from jax.experimental.pallas import tpu as pltpu
from jax.experimental.pallas import tpu_sc as plsc

_B, _C, _HD, _WD = 8, 96, 224, 224
_HW = _HD * _WD
_NW = 32
_WPB = _NW // _B
_CPW = _C // _WPB
_CROWS = 8
_CHUNK = _CROWS * _WD
_NCHUNK = _HD // _CROWS
_VPR = _WD // 16
_RSPLIT = 7
_RSUB = _HD // _RSPLIT
_ISPLIT = 4
_ISUB = _HW // _ISPLIT


def _sc_gather(v3, hp):
    mesh = plsc.VectorSubcoreMesh(core_axis_name="c", subcore_axis_name="s")

    @functools.partial(
        pl.kernel,
        mesh=mesh,
        out_type=jax.ShapeDtypeStruct((_B * _C, _HD, _WD), jnp.float32),
        scratch_types=[
            pltpu.VMEM((_HW,), jnp.int32),
            pltpu.VMEM((_HD, _WD), jnp.float32),
            [pltpu.VMEM((_CROWS, _WD), jnp.float32)] * 4,
            [pltpu.SemaphoreType.DMA] * 4,
            pltpu.SemaphoreType.DMA,
        ],
        compiler_params=pltpu.CompilerParams(
            needs_layout_passes=False, use_tc_tiling_on_sc=True
        ),
    )
    def k(v_hbm, h_hbm, out_hbm, idx_v, row_v, outs, osems, rsem):
        cid = lax.axis_index("c")
        sid = lax.axis_index("s")
        wid = sid * 2 + cid
        b = wid // _WPB
        part = wid % _WPB

        icps = [
            pltpu.async_copy(
                h_hbm.at[b, pl.ds(t * _ISUB, _ISUB)],
                idx_v.at[pl.ds(t * _ISUB, _ISUB)],
                rsem,
            )
            for t in range(_ISPLIT)
        ]
        for cp in icps:
            cp.wait()

        def gather_chunk(base, ov):
            vecs = [(orow, vcol) for orow in range(_CROWS) for vcol in range(_VPR)]
            for w0 in range(0, len(vecs), 8):
                wave = vecs[w0 : w0 + 8]
                srcs = [
                    idx_v[pl.ds(base + orow * _WD + vcol * 16, 16)]
                    for (orow, vcol) in wave
                ]
                qs = [lax.shift_right_logical(s, 8) for s in srcs]
                ms = [lax.bitwise_and(s, 255) for s in srcs]
                vals = [
                    plsc.load_gather(row_v, [q, m]) for q, m in zip(qs, ms)
                ]
                for (orow, vcol), v in zip(wave, vals):
                    ov[orow, pl.ds(vcol * 16, 16)] = v

        def chan_body(j, carry):
            r = b * _C + part * _CPW + j
            rcps = [
                pltpu.async_copy(
                    v_hbm.at[r, pl.ds(t * _RSUB, _RSUB), :],
                    row_v.at[pl.ds(t * _RSUB, _RSUB), :],
                    rsem,
                )
                for t in range(_RSPLIT)
            ]
            for cp in rcps:
                cp.wait()

            def chunk_body(t, carry2):
                first = jnp.logical_and(j == 0, t == 0)
                for u in range(4):
                    kk = t * 4 + u

                    @pl.when(jnp.logical_not(first))
                    def _():
                        pltpu.make_async_copy(
                            outs[u],
                            out_hbm.at[r, pl.ds(kk * _CROWS, _CROWS), :],
                            osems[u],
                        ).wait()

                    gather_chunk(kk * _CHUNK, outs[u])
                    pltpu.async_copy(
                        outs[u],
                        out_hbm.at[r, pl.ds(kk * _CROWS, _CROWS), :],
                        osems[u],
                    )
                return carry2

            lax.fori_loop(0, _NCHUNK // 4, chunk_body, 0)
            return carry

        lax.fori_loop(0, _CPW, chan_body, 0)

        r_last = b * _C + part * _CPW + (_CPW - 1)
        for u in range(4):
            kk = _NCHUNK - 4 + u
            pltpu.make_async_copy(
                outs[u],
                out_hbm.at[r_last, pl.ds(kk * _CROWS, _CROWS), :],
                osems[u],
            ).wait()

    return k(v3, hp)


def kernel(V, H):
    b, c, hd, wd = V.shape
    v3 = V.reshape(b * c, hd, wd)
    hp = jnp.left_shift(H // wd, 8) | (H % wd)
    out = _sc_gather(v3, hp)
    return out.reshape(b, c, hd, wd)


# speedup vs baseline: 2.0963x; 2.0963x over previous
import functools

import jax
import jax.numpy as jnp
from jax import lax
from jax.experimental import pallas as pl
from jax.experimental.pallas import tpu as pltpu
from jax.experimental.pallas import tpu_sc as plsc

_B, _C, _HD, _WD = 8, 96, 224, 224
_HW = _HD * _WD
_NW = 32
_WPB = _NW // _B
_CPW = _C // _WPB
_CROWS = 8
_CHUNK = _CROWS * _WD
_NCHUNK = _HD // _CROWS
_VPR = _WD // 16
_RSPLIT = 4
_RSUB = _HD // _RSPLIT
_ISPLIT = 4
_ISUB = _HW // _ISPLIT


def _sc_gather(v3, hp):
    mesh = plsc.VectorSubcoreMesh(core_axis_name="c", subcore_axis_name="s")

    @functools.partial(
        pl.kernel,
        mesh=mesh,
        out_type=jax.ShapeDtypeStruct((_B * _C, _HD, _WD), jnp.float32),
        scratch_types=[
            pltpu.VMEM((_HW,), jnp.int32),
            pltpu.VMEM((_HD, _WD), jnp.float32),
            [pltpu.VMEM((_CROWS, _WD), jnp.float32)] * 2,
            [pltpu.SemaphoreType.DMA] * 2,
            pltpu.SemaphoreType.DMA,
        ],
        compiler_params=pltpu.CompilerParams(
            needs_layout_passes=False, use_tc_tiling_on_sc=True
        ),
    )
    def k(v_hbm, h_hbm, out_hbm, idx_v, row_v, outs, osems, rsem):
        cid = lax.axis_index("c")
        sid = lax.axis_index("s")
        wid = sid * 2 + cid
        b = wid // _WPB
        part = wid % _WPB

        icps = [
            pltpu.async_copy(
                h_hbm.at[b, pl.ds(t * _ISUB, _ISUB)],
                idx_v.at[pl.ds(t * _ISUB, _ISUB)],
                rsem,
            )
            for t in range(_ISPLIT)
        ]
        for cp in icps:
            cp.wait()

        def gather_chunk(base, ov):
            vecs = [(orow, vcol) for orow in range(_CROWS) for vcol in range(_VPR)]
            for w0 in range(0, len(vecs), 8):
                wave = vecs[w0 : w0 + 8]
                srcs = [
                    idx_v[pl.ds(base + orow * _WD + vcol * 16, 16)]
                    for (orow, vcol) in wave
                ]
                qs = [lax.shift_right_logical(s, 8) for s in srcs]
                ms = [lax.bitwise_and(s, 255) for s in srcs]
                vals = [
                    plsc.load_gather(row_v, [q, m]) for q, m in zip(qs, ms)
                ]
                for (orow, vcol), v in zip(wave, vals):
                    ov[orow, pl.ds(vcol * 16, 16)] = v

        def chan_body(j, carry):
            r = b * _C + part * _CPW + j
            rcps = [
                pltpu.async_copy(
                    v_hbm.at[r, pl.ds(t * _RSUB, _RSUB), :],
                    row_v.at[pl.ds(t * _RSUB, _RSUB), :],
                    rsem,
                )
                for t in range(_RSPLIT)
            ]
            for cp in rcps:
                cp.wait()

            def chunk_body(t, carry2):
                first = jnp.logical_and(j == 0, t == 0)
                for u in range(2):
                    kk = t * 2 + u

                    @pl.when(jnp.logical_not(first))
                    def _():
                        pltpu.make_async_copy(
                            outs[u],
                            out_hbm.at[r, pl.ds(kk * _CROWS, _CROWS), :],
                            osems[u],
                        ).wait()

                    gather_chunk(kk * _CHUNK, outs[u])
                    pltpu.async_copy(
                        outs[u],
                        out_hbm.at[r, pl.ds(kk * _CROWS, _CROWS), :],
                        osems[u],
                    )
                return carry2

            lax.fori_loop(0, _NCHUNK // 2, chunk_body, 0)
            return carry

        lax.fori_loop(0, _CPW, chan_body, 0)

        r_last = b * _C + part * _CPW + (_CPW - 1)
        for u in range(2):
            kk = _NCHUNK - 2 + u
            pltpu.make_async_copy(
                outs[u],
                out_hbm.at[r_last, pl.ds(kk * _CROWS, _CROWS), :],
                osems[u],
            ).wait()

    return k(v3, hp)


def kernel(V, H):
    b, c, hd, wd = V.shape
    v3 = V.reshape(b * c, hd, wd)
    hp = jnp.left_shift(H // wd, 8) | (H % wd)
    out = _sc_gather(v3, hp)
    return out.reshape(b, c, hd, wd)


# speedup vs baseline: 2.0997x; 1.0016x over previous
import functools

import jax
import jax.numpy as jnp
from jax import lax
from jax.experimental import pallas as pl
from jax.experimental.pallas import tpu as pltpu
from jax.experimental.pallas import tpu_sc as plsc

_B, _C, _HD, _WD = 8, 96, 224, 224
_HW = _HD * _WD
_NW = 32
_WPB = _NW // _B
_CPW = _C // _WPB
_CROWS = 8
_CHUNK = _CROWS * _WD
_NCHUNK = _HD // _CROWS
_VPR = _WD // 16
_RSPLIT = 7
_RSUB = _HD // _RSPLIT
_ISPLIT = 4
_ISUB = _HW // _ISPLIT


def _sc_gather(v3, hp):
    mesh = plsc.VectorSubcoreMesh(core_axis_name="c", subcore_axis_name="s")

    @functools.partial(
        pl.kernel,
        mesh=mesh,
        out_type=jax.ShapeDtypeStruct((_B * _C, _HD, _WD), jnp.float32),
        scratch_types=[
            pltpu.VMEM((_HW,), jnp.int32),
            pltpu.VMEM((_HD, _WD), jnp.float32),
            [pltpu.VMEM((_CROWS, _WD), jnp.float32)] * 2,
            [pltpu.SemaphoreType.DMA] * 2,
            pltpu.SemaphoreType.DMA,
        ],
        compiler_params=pltpu.CompilerParams(
            needs_layout_passes=False, use_tc_tiling_on_sc=True
        ),
    )
    def k(v_hbm, h_hbm, out_hbm, idx_v, row_v, outs, osems, rsem):
        cid = lax.axis_index("c")
        sid = lax.axis_index("s")
        wid = sid * 2 + cid
        b = wid // _WPB
        part = wid % _WPB

        icps = [
            pltpu.async_copy(
                h_hbm.at[b, pl.ds(t * _ISUB, _ISUB)],
                idx_v.at[pl.ds(t * _ISUB, _ISUB)],
                rsem,
            )
            for t in range(_ISPLIT)
        ]
        for cp in icps:
            cp.wait()

        def gather_chunk(base, ov):
            vecs = [(orow, vcol) for orow in range(_CROWS) for vcol in range(_VPR)]
            for w0 in range(0, len(vecs), 8):
                wave = vecs[w0 : w0 + 8]
                srcs = [
                    idx_v[pl.ds(base + orow * _WD + vcol * 16, 16)]
                    for (orow, vcol) in wave
                ]
                qs = [lax.shift_right_logical(s, 8) for s in srcs]
                ms = [lax.bitwise_and(s, 255) for s in srcs]
                vals = [
                    plsc.load_gather(row_v, [q, m]) for q, m in zip(qs, ms)
                ]
                for (orow, vcol), v in zip(wave, vals):
                    ov[orow, pl.ds(vcol * 16, 16)] = v

        def chan_body(j, carry):
            r = b * _C + part * _CPW + j
            rcps = [
                pltpu.async_copy(
                    v_hbm.at[r, pl.ds(t * _RSUB, _RSUB), :],
                    row_v.at[pl.ds(t * _RSUB, _RSUB), :],
                    rsem,
                )
                for t in range(_RSPLIT)
            ]
            for cp in rcps:
                cp.wait()

            def chunk_body(t, carry2):
                first = jnp.logical_and(j == 0, t == 0)
                for u in range(2):
                    kk = t * 2 + u

                    @pl.when(jnp.logical_not(first))
                    def _():
                        pltpu.make_async_copy(
                            outs[u],
                            out_hbm.at[r, pl.ds(kk * _CROWS, _CROWS), :],
                            osems[u],
                        ).wait()

                    gather_chunk(kk * _CHUNK, outs[u])
                    pltpu.async_copy(
                        outs[u],
                        out_hbm.at[r, pl.ds(kk * _CROWS, _CROWS), :],
                        osems[u],
                    )
                return carry2

            lax.fori_loop(0, _NCHUNK // 2, chunk_body, 0)
            return carry

        lax.fori_loop(0, _CPW, chan_body, 0)

        r_last = b * _C + part * _CPW + (_CPW - 1)
        for u in range(2):
            kk = _NCHUNK - 2 + u
            pltpu.make_async_copy(
                outs[u],
                out_hbm.at[r_last, pl.ds(kk * _CROWS, _CROWS), :],
                osems[u],
            ).wait()

    return k(v3, hp)


def kernel(V, H):
    b, c, hd, wd = V.shape
    v3 = V.reshape(b * c, hd, wd)
    hp = jnp.left_shift(H // wd, 8) | (H % wd)
    out = _sc_gather(v3, hp)
    return out.reshape(b, c, hd, wd)
